# Initial kernel scaffold; baseline (speedup 1.0000x reference)
import jax
import jax.numpy as jnp
from jax.experimental import pallas as pl


def kernel(item_ids, frozen_emb, item_tags, item_emb_w, tag_emb_w):
    raise NotImplementedError("write your pallas kernel here")


# speedup vs baseline: 8.2031x; 8.2031x over previous
import functools

import jax
import jax.numpy as jnp
from jax import lax
from jax.experimental import pallas as pl
from jax.experimental.pallas import tpu as pltpu
from jax.experimental.pallas import tpu_sc as plsc

_NUM_ITEMS = 100000
_EMBED_DIM = 64
_FROZEN_DIM = 128
_NUM_TAGS = 1000
_TAG_DIM = 16
_TAGS_PER_ITEM = 5
_TAG_PAD = 16

_NC = 2
_NS = 16
_NW = _NC * _NS

_N = 4096 * 50
_PER_W = _N // _NW
_C = 128
_NCHUNK = _PER_W // _C
_OUT_DIM = _FROZEN_DIM + _EMBED_DIM + _TAG_DIM
_L = 16


def _sc_body(ids_hbm, frozen_hbm, tagspad_hbm, item_w_hbm, tag_w_hbm, out_hbm,
             idx_c, frozen_c, learn_c, tags_c, tagmean_c, tag_tab,
             sem0, sem1, sem2):
    wid = lax.axis_index("s") * _NC + lax.axis_index("c")
    base = wid * _PER_W

    pltpu.sync_copy(tag_w_hbm, tag_tab)

    iota = lax.iota(jnp.int32, _L)
    inv_tags = jnp.float32(1.0 / _TAGS_PER_ITEM)

    def chunk(g, carry):
        row0 = base + g * _C
        pltpu.sync_copy(ids_hbm.at[pl.ds(row0, _C)], idx_c)
        cp0 = pltpu.async_copy(frozen_hbm.at[idx_c], frozen_c, sem0)
        cp1 = pltpu.async_copy(item_w_hbm.at[idx_c], learn_c, sem1)
        cp2 = pltpu.async_copy(tagspad_hbm.at[idx_c], tags_c, sem2)
        cp2.wait()

        def grp(h, c2):
            rows = h * _L + iota
            tv = [plsc.load_gather(tags_c, [rows, jnp.full((_L,), t, jnp.int32)])
                  for t in range(_TAGS_PER_ITEM)]
            for d in range(_TAG_DIM):
                dcol = jnp.full((_L,), d, jnp.int32)
                acc = plsc.load_gather(tag_tab, [tv[0], dcol])
                for t in range(1, _TAGS_PER_ITEM):
                    acc = acc + plsc.load_gather(tag_tab, [tv[t], dcol])
                plsc.store_scatter(tagmean_c, [rows, dcol], acc * inv_tags)
            return c2

        lax.fori_loop(0, _C // _L, grp, 0)

        cp0.wait()
        cp1.wait()
        pltpu.sync_copy(frozen_c, out_hbm.at[pl.ds(row0, _C), pl.ds(0, _FROZEN_DIM)])
        pltpu.sync_copy(learn_c, out_hbm.at[pl.ds(row0, _C), pl.ds(_FROZEN_DIM, _EMBED_DIM)])
        pltpu.sync_copy(tagmean_c,
                        out_hbm.at[pl.ds(row0, _C), pl.ds(_FROZEN_DIM + _EMBED_DIM, _TAG_DIM)])
        return carry

    lax.fori_loop(0, _NCHUNK, chunk, 0)


_sc_kernel = functools.partial(
    pl.kernel,
    out_type=jax.ShapeDtypeStruct((_N, _OUT_DIM), jnp.float32),
    mesh=plsc.VectorSubcoreMesh(core_axis_name="c", subcore_axis_name="s",
                                num_cores=_NC, num_subcores=_NS),
    scratch_types=[
        pltpu.VMEM((_C,), jnp.int32),
        pltpu.VMEM((_C, _FROZEN_DIM), jnp.float32),
        pltpu.VMEM((_C, _EMBED_DIM), jnp.float32),
        pltpu.VMEM((_C, _TAG_PAD), jnp.int32),
        pltpu.VMEM((_C, _TAG_DIM), jnp.float32),
        pltpu.VMEM((_NUM_TAGS, _TAG_DIM), jnp.float32),
        pltpu.SemaphoreType.DMA,
        pltpu.SemaphoreType.DMA,
        pltpu.SemaphoreType.DMA,
    ],
    compiler_params=pltpu.CompilerParams(use_tc_tiling_on_sc=False,
                                         needs_layout_passes=False),
)(_sc_body)


def kernel(item_ids, frozen_emb, item_tags, item_emb_w, tag_emb_w):
    batch, hist = item_ids.shape
    ids_flat = item_ids.reshape(-1).astype(jnp.int32)
    tags_pad = jnp.zeros((_NUM_ITEMS, _TAG_PAD), jnp.int32)
    tags_pad = tags_pad.at[:, :_TAGS_PER_ITEM].set(item_tags.astype(jnp.int32))
    out = _sc_kernel(ids_flat, frozen_emb, tags_pad, item_emb_w, tag_emb_w)
    return out.reshape(batch, hist, _OUT_DIM)


# speedup vs baseline: 9.0994x; 1.1093x over previous
import functools

import jax
import jax.numpy as jnp
from jax import lax
from jax.experimental import pallas as pl
from jax.experimental.pallas import tpu as pltpu
from jax.experimental.pallas import tpu_sc as plsc

_NUM_ITEMS = 100000
_EMBED_DIM = 64
_FROZEN_DIM = 128
_NUM_TAGS = 1000
_TAG_DIM = 16
_TAGS_PER_ITEM = 5
_TAG_PAD = 16

_NC = 2
_NS = 16
_NW = _NC * _NS

_N = 4096 * 50
_PER_W = _N // _NW
_C = 128
_NCHUNK = _PER_W // _C
_NSTEP = _NCHUNK // 2
_OUT_DIM = _FROZEN_DIM + _EMBED_DIM + _TAG_DIM
_L = 16


def _sc_body(ids_hbm, frozen_hbm, tagspad_hbm, item_w_hbm, tag_w_hbm, out_hbm,
             idxs_v, frozen0, frozen1, learn0, learn1, tags0, tags1,
             tagmean0, tagmean1, tag_tab, gsem0, gsem1, wsem0, wsem1):
    wid = lax.axis_index("s") * _NC + lax.axis_index("c")
    base = wid * _PER_W

    frozen_c = (frozen0, frozen1)
    learn_c = (learn0, learn1)
    tags_c = (tags0, tags1)
    tagmean_c = (tagmean0, tagmean1)
    gsem = (gsem0, gsem1)
    wsem = (wsem0, wsem1)

    pltpu.sync_copy(tag_w_hbm, tag_tab)
    pltpu.sync_copy(ids_hbm.at[pl.ds(wid * _NCHUNK, _NCHUNK)], idxs_v)

    iota = lax.iota(jnp.int32, _L)
    inv_tags = jnp.float32(1.0 / _TAGS_PER_ITEM)

    def ig(g, b):
        idxr = idxs_v.at[g]
        pltpu.async_copy(frozen_hbm.at[idxr], frozen_c[b], gsem[b])
        pltpu.async_copy(item_w_hbm.at[idxr], learn_c[b], gsem[b])
        pltpu.async_copy(tagspad_hbm.at[idxr], tags_c[b], gsem[b])

    def waitg(b):
        idxr = idxs_v.at[0]
        pltpu.make_async_copy(frozen_hbm.at[idxr], frozen_c[b], gsem[b]).wait()
        pltpu.make_async_copy(item_w_hbm.at[idxr], learn_c[b], gsem[b]).wait()
        pltpu.make_async_copy(tagspad_hbm.at[idxr], tags_c[b], gsem[b]).wait()

    def iw(g, b):
        row0 = base + g * _C
        pltpu.async_copy(frozen_c[b],
                         out_hbm.at[pl.ds(row0, _C), pl.ds(0, _FROZEN_DIM)],
                         wsem[b])
        pltpu.async_copy(learn_c[b],
                         out_hbm.at[pl.ds(row0, _C), pl.ds(_FROZEN_DIM, _EMBED_DIM)],
                         wsem[b])
        pltpu.async_copy(tagmean_c[b],
                         out_hbm.at[pl.ds(row0, _C),
                                    pl.ds(_FROZEN_DIM + _EMBED_DIM, _TAG_DIM)],
                         wsem[b])

    def waitw(b):
        pltpu.make_async_copy(frozen_c[b],
                              out_hbm.at[pl.ds(base, _C), pl.ds(0, _FROZEN_DIM)],
                              wsem[b]).wait()
        pltpu.make_async_copy(learn_c[b],
                              out_hbm.at[pl.ds(base, _C), pl.ds(_FROZEN_DIM, _EMBED_DIM)],
                              wsem[b]).wait()
        pltpu.make_async_copy(tagmean_c[b],
                              out_hbm.at[pl.ds(base, _C),
                                         pl.ds(_FROZEN_DIM + _EMBED_DIM, _TAG_DIM)],
                              wsem[b]).wait()

    def compute(b):
        tg, tm = tags_c[b], tagmean_c[b]

        def grp(h, c2):
            rows = h * _L + iota
            tv = [plsc.load_gather(tg, [rows, jnp.full((_L,), t, jnp.int32)])
                  for t in range(_TAGS_PER_ITEM)]
            for d in range(_TAG_DIM):
                dcol = jnp.full((_L,), d, jnp.int32)
                acc = plsc.load_gather(tag_tab, [tv[0], dcol])
                for t in range(1, _TAGS_PER_ITEM):
                    acc = acc + plsc.load_gather(tag_tab, [tv[t], dcol])
                plsc.store_scatter(tm, [rows, dcol], acc * inv_tags)
            return c2

        lax.fori_loop(0, _C // _L, grp, 0)

    ig(0, 0)

    def step(s, carry):
        g0 = 2 * s
        pl.when(s > 0)(lambda: waitw(1))
        ig(g0 + 1, 1)
        waitg(0)
        compute(0)
        iw(g0, 0)

        def prefetch_even():
            waitw(0)
            ig(g0 + 2, 0)

        pl.when(s < _NSTEP - 1)(prefetch_even)
        waitg(1)
        compute(1)
        iw(g0 + 1, 1)
        return carry

    lax.fori_loop(0, _NSTEP, step, 0)
    waitw(0)
    waitw(1)


_sc_kernel = functools.partial(
    pl.kernel,
    out_type=jax.ShapeDtypeStruct((_N, _OUT_DIM), jnp.float32),
    mesh=plsc.VectorSubcoreMesh(core_axis_name="c", subcore_axis_name="s",
                                num_cores=_NC, num_subcores=_NS),
    scratch_types=[
        pltpu.VMEM((_NCHUNK, _C), jnp.int32),
        pltpu.VMEM((_C, _FROZEN_DIM), jnp.float32),
        pltpu.VMEM((_C, _FROZEN_DIM), jnp.float32),
        pltpu.VMEM((_C, _EMBED_DIM), jnp.float32),
        pltpu.VMEM((_C, _EMBED_DIM), jnp.float32),
        pltpu.VMEM((_C, _TAG_PAD), jnp.int32),
        pltpu.VMEM((_C, _TAG_PAD), jnp.int32),
        pltpu.VMEM((_C, _TAG_DIM), jnp.float32),
        pltpu.VMEM((_C, _TAG_DIM), jnp.float32),
        pltpu.VMEM((_NUM_TAGS, _TAG_DIM), jnp.float32),
        pltpu.SemaphoreType.DMA,
        pltpu.SemaphoreType.DMA,
        pltpu.SemaphoreType.DMA,
        pltpu.SemaphoreType.DMA,
    ],
    compiler_params=pltpu.CompilerParams(use_tc_tiling_on_sc=False,
                                         needs_layout_passes=False),
)(_sc_body)


def kernel(item_ids, frozen_emb, item_tags, item_emb_w, tag_emb_w):
    batch, hist = item_ids.shape
    ids_chunks = item_ids.reshape(_NW * _NCHUNK, _C).astype(jnp.int32)
    tags_pad = jnp.zeros((_NUM_ITEMS, _TAG_PAD), jnp.int32)
    tags_pad = tags_pad.at[:, :_TAGS_PER_ITEM].set(item_tags.astype(jnp.int32))
    out = _sc_kernel(ids_chunks, frozen_emb, tags_pad, item_emb_w, tag_emb_w)
    return out.reshape(batch, hist, _OUT_DIM)


# speedup vs baseline: 10.7979x; 1.1867x over previous
import functools

import jax
import jax.numpy as jnp
from jax import lax
from jax.experimental import pallas as pl
from jax.experimental.pallas import tpu as pltpu
from jax.experimental.pallas import tpu_sc as plsc

_NUM_ITEMS = 100000
_EMBED_DIM = 64
_FROZEN_DIM = 128
_NUM_TAGS = 1000
_TAG_DIM = 16
_TAGS_PER_ITEM = 5
_TAG_PAD = 8

_NC = 2
_NS = 16
_NW = _NC * _NS

_B = 4096
_H = 50
_C = 128
_NBT = _B // _C
_NSLAB = _H * _NBT
_PER_W = _NSLAB // _NW
_NSTEP = _PER_W // 2
_OUT_DIM = _FROZEN_DIM + _EMBED_DIM + _TAG_DIM
_NDT = _OUT_DIM // 8
_PITCH = _C + 1
_L = 16


def _sc_body(ids_hbm, frozen_hbm, tags_hbm, item_w_hbm, tag_w_hbm, w_hbm,
             idxs_v, frozen0, frozen1, learn0, learn1, tags0, tags1,
             slab0, slab1, tag_tab, gsem0, gsem1, wsem0, wsem1):
    wid = lax.axis_index("s") * _NC + lax.axis_index("c")
    base = wid * _PER_W

    frozen_c = (frozen0, frozen1)
    learn_c = (learn0, learn1)
    tags_c = (tags0, tags1)
    slab_c = (slab0, slab1)
    gsem = (gsem0, gsem1)
    wsem = (wsem0, wsem1)

    pltpu.sync_copy(tag_w_hbm, tag_tab)
    pltpu.sync_copy(ids_hbm.at[pl.ds(base, _PER_W)], idxs_v)

    iota = lax.iota(jnp.int32, _L)
    inv_tags = jnp.float32(1.0 / _TAGS_PER_ITEM)

    dtv = []
    drv = []
    for k in range(12):
        d = k * _L + iota
        dtv.append(lax.shift_right_logical(d, 3))
        drv.append(lax.bitwise_and(d, 7))

    def ig(g, b):
        idxr = idxs_v.at[g]
        pltpu.async_copy(frozen_hbm.at[idxr], frozen_c[b], gsem[b])
        pltpu.async_copy(item_w_hbm.at[idxr], learn_c[b], gsem[b])
        pltpu.async_copy(tags_hbm.at[idxr], tags_c[b], gsem[b])

    def waitg(b):
        idxr = idxs_v.at[0]
        pltpu.make_async_copy(frozen_hbm.at[idxr], frozen_c[b], gsem[b]).wait()
        pltpu.make_async_copy(item_w_hbm.at[idxr], learn_c[b], gsem[b]).wait()
        pltpu.make_async_copy(tags_hbm.at[idxr], tags_c[b], gsem[b]).wait()

    def iw(g, b):
        s = base + g
        l = s // _NBT
        bt = s % _NBT
        pltpu.async_copy(slab_c[b].at[:, :, pl.ds(0, _C)],
                         w_hbm.at[l, :, bt], wsem[b])

    def waitw(b):
        pltpu.make_async_copy(slab_c[b].at[:, :, pl.ds(0, _C)],
                              w_hbm.at[0, :, 0], wsem[b]).wait()

    def compute(b):
        fz, ln, tg, W = frozen_c[b], learn_c[b], tags_c[b], slab_c[b]

        def item(br, c):
            col = jnp.full((_L,), 0, jnp.int32) + br
            for k in range(8):
                v = fz[br, pl.ds(k * _L, _L)]
                plsc.store_scatter(W, [dtv[k], drv[k], col], v)
            for k in range(4):
                v = ln[br, pl.ds(k * _L, _L)]
                plsc.store_scatter(W, [dtv[8 + k], drv[8 + k], col], v)
            return c

        lax.fori_loop(0, _C, item, 0)

        def grp(h, c):
            rows = h * _L + iota
            tv = [plsc.load_gather(tg, [rows, jnp.full((_L,), t, jnp.int32)])
                  for t in range(_TAGS_PER_ITEM)]
            for d in range(_TAG_DIM):
                dcol = jnp.full((_L,), d, jnp.int32)
                acc = plsc.load_gather(tag_tab, [tv[0], dcol])
                for t in range(1, _TAGS_PER_ITEM):
                    acc = acc + plsc.load_gather(tag_tab, [tv[t], dcol])
                dd = _FROZEN_DIM + _EMBED_DIM + d
                W[dd >> 3, dd & 7, pl.ds(h * _L, _L)] = acc * inv_tags
            return c

        lax.fori_loop(0, _C // _L, grp, 0)

    ig(0, 0)

    def step(s, carry):
        g0 = 2 * s
        pl.when(s > 0)(lambda: waitw(1))
        ig(g0 + 1, 1)
        waitg(0)
        compute(0)
        iw(g0, 0)

        def prefetch_even():
            waitw(0)
            ig(g0 + 2, 0)

        pl.when(s < _NSTEP - 1)(prefetch_even)
        waitg(1)
        compute(1)
        iw(g0 + 1, 1)
        return carry

    lax.fori_loop(0, _NSTEP, step, 0)
    waitw(0)
    waitw(1)


_sc_kernel = functools.partial(
    pl.kernel,
    out_type=jax.ShapeDtypeStruct((_H, _NDT, _NBT, 8, _C), jnp.float32),
    mesh=plsc.VectorSubcoreMesh(core_axis_name="c", subcore_axis_name="s",
                                num_cores=_NC, num_subcores=_NS),
    scratch_types=[
        pltpu.VMEM((_PER_W, _C), jnp.int32),
        pltpu.VMEM((_C, _FROZEN_DIM), jnp.float32),
        pltpu.VMEM((_C, _FROZEN_DIM), jnp.float32),
        pltpu.VMEM((_C, _EMBED_DIM), jnp.float32),
        pltpu.VMEM((_C, _EMBED_DIM), jnp.float32),
        pltpu.VMEM((_C, _TAG_PAD), jnp.int32),
        pltpu.VMEM((_C, _TAG_PAD), jnp.int32),
        pltpu.VMEM((_NDT, 8, _PITCH), jnp.float32),
        pltpu.VMEM((_NDT, 8, _PITCH), jnp.float32),
        pltpu.VMEM((_NUM_TAGS, _TAG_DIM), jnp.float32),
        pltpu.SemaphoreType.DMA,
        pltpu.SemaphoreType.DMA,
        pltpu.SemaphoreType.DMA,
        pltpu.SemaphoreType.DMA,
    ],
    compiler_params=pltpu.CompilerParams(use_tc_tiling_on_sc=False,
                                         needs_layout_passes=False),
)(_sc_body)


def kernel(item_ids, frozen_emb, item_tags, item_emb_w, tag_emb_w):
    ids_t = item_ids.astype(jnp.int32).T.reshape(_NSLAB, _C)
    tags8 = jnp.pad(item_tags.astype(jnp.int32),
                    ((0, 0), (0, _TAG_PAD - _TAGS_PER_ITEM)))
    w = _sc_kernel(ids_t, frozen_emb, tags8, item_emb_w, tag_emb_w)
    return jnp.transpose(w, (2, 4, 0, 1, 3)).reshape(_B, _H, _OUT_DIM)


# speedup vs baseline: 10.9894x; 1.0177x over previous
import functools

import jax
import jax.numpy as jnp
from jax import lax
from jax.experimental import pallas as pl
from jax.experimental.pallas import tpu as pltpu
from jax.experimental.pallas import tpu_sc as plsc

_NUM_ITEMS = 100000
_EMBED_DIM = 64
_FROZEN_DIM = 128
_NUM_TAGS = 1000
_TAG_DIM = 16
_TAGS_PER_ITEM = 5
_TAG_PAD = 8

_NC = 2
_NS = 16
_NW = _NC * _NS

_B = 4096
_H = 50
_C = 128
_NBT = _B // _C
_NSLAB = _H * _NBT
_PER_W = _NSLAB // _NW
_NSTEP = _PER_W // 2
_OUT_DIM = _FROZEN_DIM + _EMBED_DIM + _TAG_DIM
_NDT = _OUT_DIM // 8
_PITCH = _C + 1
_L = 16


def _sc_body(ids_hbm, frozen_hbm, tags_hbm, item_w_hbm, tag_w_hbm, w_hbm,
             idxs_v, frozen0, frozen1, learn0, learn1, tags0, tags1,
             slab0, slab1, tag_tab, gsem0, gsem1, wsem0, wsem1):
    wid = lax.axis_index("s") * _NC + lax.axis_index("c")
    base = wid * _PER_W

    frozen_c = (frozen0, frozen1)
    learn_c = (learn0, learn1)
    tags_c = (tags0, tags1)
    slab_c = (slab0, slab1)
    gsem = (gsem0, gsem1)
    wsem = (wsem0, wsem1)

    pltpu.sync_copy(tag_w_hbm, tag_tab)
    pltpu.sync_copy(ids_hbm.at[pl.ds(base, _PER_W)], idxs_v)

    iota = lax.iota(jnp.int32, _L)
    inv_tags = jnp.float32(1.0 / _TAGS_PER_ITEM)

    dtv = []
    drv = []
    for k in range(12):
        d = k * _L + iota
        dtv.append(lax.shift_right_logical(d, 3))
        drv.append(lax.bitwise_and(d, 7))

    def ig(g, b):
        idxr = idxs_v.at[g]
        pltpu.async_copy(frozen_hbm.at[idxr], frozen_c[b], gsem[b])
        pltpu.async_copy(item_w_hbm.at[idxr], learn_c[b], gsem[b])
        pltpu.async_copy(tags_hbm.at[idxr], tags_c[b], gsem[b])

    def waitg(b):
        idxr = idxs_v.at[0]
        pltpu.make_async_copy(frozen_hbm.at[idxr], frozen_c[b], gsem[b]).wait()
        pltpu.make_async_copy(item_w_hbm.at[idxr], learn_c[b], gsem[b]).wait()
        pltpu.make_async_copy(tags_hbm.at[idxr], tags_c[b], gsem[b]).wait()

    def iw(g, b):
        s = base + g
        l = s // _NBT
        bt = s % _NBT
        pltpu.async_copy(slab_c[b].at[:, :, pl.ds(0, _C)],
                         w_hbm.at[l, :, bt], wsem[b])

    def waitw(b):
        pltpu.make_async_copy(slab_c[b].at[:, :, pl.ds(0, _C)],
                              w_hbm.at[0, :, 0], wsem[b]).wait()

    def compute(b):
        fz, ln, tg, W = frozen_c[b], learn_c[b], tags_c[b], slab_c[b]

        def item4(i, c):
            br0 = i * 4
            for j in range(4):
                br = br0 + j
                col = jnp.full((_L,), 0, jnp.int32) + br
                for k in range(8):
                    v = fz[br, pl.ds(k * _L, _L)]
                    plsc.store_scatter(W, [dtv[k], drv[k], col], v)
                for k in range(4):
                    v = ln[br, pl.ds(k * _L, _L)]
                    plsc.store_scatter(W, [dtv[8 + k], drv[8 + k], col], v)
            return c

        lax.fori_loop(0, _C // 4, item4, 0)

        def grp2(h2, c):
            for j in range(2):
                h = h2 * 2 + j
                rows = h * _L + iota
                tv = [plsc.load_gather(tg, [rows, jnp.full((_L,), t, jnp.int32)])
                      for t in range(_TAGS_PER_ITEM)]
                for d in range(_TAG_DIM):
                    dcol = jnp.full((_L,), d, jnp.int32)
                    acc = plsc.load_gather(tag_tab, [tv[0], dcol])
                    for t in range(1, _TAGS_PER_ITEM):
                        acc = acc + plsc.load_gather(tag_tab, [tv[t], dcol])
                    dd = _FROZEN_DIM + _EMBED_DIM + d
                    W[dd >> 3, dd & 7, pl.ds(h * _L, _L)] = acc * inv_tags
            return c

        lax.fori_loop(0, _C // _L // 2, grp2, 0)

    ig(0, 0)

    def step(s, carry):
        g0 = 2 * s
        pl.when(s > 0)(lambda: waitw(1))
        ig(g0 + 1, 1)
        waitg(0)
        compute(0)
        iw(g0, 0)

        def prefetch_even():
            waitw(0)
            ig(g0 + 2, 0)

        pl.when(s < _NSTEP - 1)(prefetch_even)
        waitg(1)
        compute(1)
        iw(g0 + 1, 1)
        return carry

    lax.fori_loop(0, _NSTEP, step, 0)
    waitw(0)
    waitw(1)


_sc_kernel = functools.partial(
    pl.kernel,
    out_type=jax.ShapeDtypeStruct((_H, _NDT, _NBT, 8, _C), jnp.float32),
    mesh=plsc.VectorSubcoreMesh(core_axis_name="c", subcore_axis_name="s",
                                num_cores=_NC, num_subcores=_NS),
    scratch_types=[
        pltpu.VMEM((_PER_W, _C), jnp.int32),
        pltpu.VMEM((_C, _FROZEN_DIM), jnp.float32),
        pltpu.VMEM((_C, _FROZEN_DIM), jnp.float32),
        pltpu.VMEM((_C, _EMBED_DIM), jnp.float32),
        pltpu.VMEM((_C, _EMBED_DIM), jnp.float32),
        pltpu.VMEM((_C, _TAG_PAD), jnp.int32),
        pltpu.VMEM((_C, _TAG_PAD), jnp.int32),
        pltpu.VMEM((_NDT, 8, _PITCH), jnp.float32),
        pltpu.VMEM((_NDT, 8, _PITCH), jnp.float32),
        pltpu.VMEM((_NUM_TAGS, _TAG_DIM), jnp.float32),
        pltpu.SemaphoreType.DMA,
        pltpu.SemaphoreType.DMA,
        pltpu.SemaphoreType.DMA,
        pltpu.SemaphoreType.DMA,
    ],
    compiler_params=pltpu.CompilerParams(use_tc_tiling_on_sc=False,
                                         needs_layout_passes=False),
)(_sc_body)


def kernel(item_ids, frozen_emb, item_tags, item_emb_w, tag_emb_w):
    ids_t = item_ids.astype(jnp.int32).T.reshape(_NSLAB, _C)
    tags8 = jnp.pad(item_tags.astype(jnp.int32),
                    ((0, 0), (0, _TAG_PAD - _TAGS_PER_ITEM)))
    w = _sc_kernel(ids_t, frozen_emb, tags8, item_emb_w, tag_emb_w)
    return jnp.transpose(w, (2, 4, 0, 1, 3)).reshape(_B, _H, _OUT_DIM)


# speedup vs baseline: 17.2908x; 1.5734x over previous
import functools

import jax
import jax.numpy as jnp
from jax import lax
from jax.experimental import pallas as pl
from jax.experimental.pallas import tpu as pltpu
from jax.experimental.pallas import tpu_sc as plsc

_NUM_ITEMS = 100000
_EMBED_DIM = 64
_FROZEN_DIM = 128
_NUM_TAGS = 1000
_TAG_DIM = 16
_TAGS_PER_ITEM = 5
_TAG_PAD = 8

_NC = 2
_NS = 16
_NW = _NC * _NS

_B = 4096
_H = 50
_C = 128
_NBT = _B // _C
_NSLAB = _H * _NBT
_PER_W = _NSLAB // _NW
_NSTEP = _PER_W // 2
_OUT_DIM = _FROZEN_DIM + _EMBED_DIM + _TAG_DIM
_NDT = _OUT_DIM // 8
_PITCH = _C + 1
_L = 16


def _sc_body(ids_hbm, frozen_hbm, tags_hbm, item_w_hbm, tag_w_hbm, w_hbm,
             idx0, idx1, frozen0, frozen1, learn0, learn1, tags0, tags1,
             slab0, slab1, tag_tab, gsem0, gsem1, wsem0, wsem1):
    wid = lax.axis_index("s") * _NC + lax.axis_index("c")
    base = wid * _PER_W

    idx_c = (idx0, idx1)
    frozen_c = (frozen0, frozen1)
    learn_c = (learn0, learn1)
    tags_c = (tags0, tags1)
    slab_c = (slab0, slab1)
    gsem = (gsem0, gsem1)
    wsem = (wsem0, wsem1)

    pltpu.sync_copy(tag_w_hbm, tag_tab)

    iota = lax.iota(jnp.int32, _L)
    inv_tags = jnp.float32(1.0 / _TAGS_PER_ITEM)

    dtv = []
    drv = []
    for k in range(12):
        d = k * _L + iota
        dtv.append(lax.shift_right_logical(d, 3))
        drv.append(lax.bitwise_and(d, 7))

    def ig(g, b):
        pltpu.sync_copy(ids_hbm.at[base + g], idx_c[b])
        idxr = idx_c[b]
        pltpu.async_copy(frozen_hbm.at[idxr], frozen_c[b], gsem[b])
        pltpu.async_copy(item_w_hbm.at[idxr], learn_c[b], gsem[b])
        pltpu.async_copy(tags_hbm.at[idxr], tags_c[b], gsem[b])

    def waitg(b):
        idxr = idx_c[b]
        pltpu.make_async_copy(frozen_hbm.at[idxr], frozen_c[b], gsem[b]).wait()
        pltpu.make_async_copy(item_w_hbm.at[idxr], learn_c[b], gsem[b]).wait()
        pltpu.make_async_copy(tags_hbm.at[idxr], tags_c[b], gsem[b]).wait()

    def iw(g, b):
        s = base + g
        l = s // _NBT
        bt = s % _NBT
        pltpu.async_copy(slab_c[b].at[:, :, pl.ds(0, _C)],
                         w_hbm.at[l, :, bt], wsem[b])

    def waitw(b):
        pltpu.make_async_copy(slab_c[b].at[:, :, pl.ds(0, _C)],
                              w_hbm.at[0, :, 0], wsem[b]).wait()

    def compute(b):
        fz, ln, tg, W = frozen_c[b], learn_c[b], tags_c[b], slab_c[b]

        @plsc.parallel_loop(0, _C, 1, unroll=4)
        def _item(br):
            col = jnp.full((_L,), 0, jnp.int32) + br
            for k in range(8):
                v = fz[br, pl.ds(k * _L, _L)]
                plsc.store_scatter(W, [dtv[k], drv[k], col], v)
            for k in range(4):
                v = ln[br, pl.ds(k * _L, _L)]
                plsc.store_scatter(W, [dtv[8 + k], drv[8 + k], col], v)

        @plsc.parallel_loop(0, _C // _L, 1, unroll=2)
        def _grp(h):
            rows = h * _L + iota
            tv = [plsc.load_gather(tg, [rows, jnp.full((_L,), t, jnp.int32)])
                  for t in range(_TAGS_PER_ITEM)]
            for d in range(_TAG_DIM):
                dcol = jnp.full((_L,), d, jnp.int32)
                acc = plsc.load_gather(tag_tab, [tv[0], dcol])
                for t in range(1, _TAGS_PER_ITEM):
                    acc = acc + plsc.load_gather(tag_tab, [tv[t], dcol])
                dd = _FROZEN_DIM + _EMBED_DIM + d
                W[dd >> 3, dd & 7, pl.ds(h * _L, _L)] = acc * inv_tags

    ig(0, 0)

    def step(s, carry):
        g0 = 2 * s
        pl.when(s > 0)(lambda: waitw(1))
        ig(g0 + 1, 1)
        waitg(0)
        compute(0)
        iw(g0, 0)

        def prefetch_even():
            waitw(0)
            ig(g0 + 2, 0)

        pl.when(s < _NSTEP - 1)(prefetch_even)
        waitg(1)
        compute(1)
        iw(g0 + 1, 1)
        return carry

    lax.fori_loop(0, _NSTEP, step, 0)
    waitw(0)
    waitw(1)


_sc_kernel = functools.partial(
    pl.kernel,
    out_type=jax.ShapeDtypeStruct((_H, _NDT, _NBT, 8, _C), jnp.float32),
    mesh=plsc.VectorSubcoreMesh(core_axis_name="c", subcore_axis_name="s",
                                num_cores=_NC, num_subcores=_NS),
    scratch_types=[
        pltpu.VMEM((_C,), jnp.int32),
        pltpu.VMEM((_C,), jnp.int32),
        pltpu.VMEM((_C, _FROZEN_DIM), jnp.float32),
        pltpu.VMEM((_C, _FROZEN_DIM), jnp.float32),
        pltpu.VMEM((_C, _EMBED_DIM), jnp.float32),
        pltpu.VMEM((_C, _EMBED_DIM), jnp.float32),
        pltpu.VMEM((_C, _TAG_PAD), jnp.int32),
        pltpu.VMEM((_C, _TAG_PAD), jnp.int32),
        pltpu.VMEM((_NDT, 8, _PITCH), jnp.float32),
        pltpu.VMEM((_NDT, 8, _PITCH), jnp.float32),
        pltpu.VMEM((_NUM_TAGS, _TAG_DIM), jnp.float32),
        pltpu.SemaphoreType.DMA,
        pltpu.SemaphoreType.DMA,
        pltpu.SemaphoreType.DMA,
        pltpu.SemaphoreType.DMA,
    ],
    compiler_params=pltpu.CompilerParams(use_tc_tiling_on_sc=False,
                                         needs_layout_passes=False),
)(_sc_body)


def kernel(item_ids, frozen_emb, item_tags, item_emb_w, tag_emb_w):
    ids_t = item_ids.astype(jnp.int32).T.reshape(_NSLAB, _C)
    tags8 = jnp.pad(item_tags.astype(jnp.int32),
                    ((0, 0), (0, _TAG_PAD - _TAGS_PER_ITEM)))
    w = _sc_kernel(ids_t, frozen_emb, tags8, item_emb_w, tag_emb_w)
    return jnp.transpose(w, (2, 4, 0, 1, 3)).reshape(_B, _H, _OUT_DIM)


# speedup vs baseline: 19.4610x; 1.1255x over previous
import functools

import jax
import jax.numpy as jnp
from jax import lax
from jax.experimental import pallas as pl
from jax.experimental.pallas import tpu as pltpu
from jax.experimental.pallas import tpu_sc as plsc

_NUM_ITEMS = 100000
_EMBED_DIM = 64
_FROZEN_DIM = 128
_NUM_TAGS = 1000
_TAG_DIM = 16
_TAGS_PER_ITEM = 5
_TAG_PACK = 2

_NC = 2
_NS = 16
_NW = _NC * _NS

_B = 4096
_H = 50
_C = 128
_NBT = _B // _C
_NSLAB = _H * _NBT
_PER_W = _NSLAB // _NW
_NSTEP = _PER_W // 2
_OUT_DIM = _FROZEN_DIM + _EMBED_DIM + _TAG_DIM
_NDT = _OUT_DIM // 8
_PITCH = _C + 1
_L = 16


def _sc_body(ids_hbm, ids4_hbm, frozen_hbm, tags_hbm, item_w_hbm, tag_w_hbm,
             w_hbm,
             idx0, idx1, idx4_0, idx4_1, frozen0, frozen1, learn0, learn1,
             tags0, tags1, slab0, slab1, tag_tab, gsem0, gsem1, wsem0, wsem1,
             isem0, isem1):
    wid = lax.axis_index("s") * _NC + lax.axis_index("c")
    base = wid * _PER_W

    idx_c = (idx0, idx1)
    idx4_c = (idx4_0, idx4_1)
    frozen_c = (frozen0, frozen1)
    learn_c = (learn0, learn1)
    tags_c = (tags0, tags1)
    slab_c = (slab0, slab1)
    gsem = (gsem0, gsem1)
    wsem = (wsem0, wsem1)
    isem = (isem0, isem1)

    pltpu.sync_copy(tag_w_hbm, tag_tab)

    iota = lax.iota(jnp.int32, _L)
    inv_tags = jnp.float32(1.0 / _TAGS_PER_ITEM)

    dtv = []
    drv = []
    for k in range(12):
        d = k * _L + iota
        dtv.append(lax.shift_right_logical(d, 3))
        drv.append(lax.bitwise_and(d, 7))

    def idx_start(g, b):
        pltpu.async_copy(ids_hbm.at[base + g], idx_c[b], isem[b])
        pltpu.async_copy(ids4_hbm.at[base + g], idx4_c[b], isem[b])

    def ig(g, b):
        pltpu.make_async_copy(ids_hbm.at[base], idx_c[b], isem[b]).wait()
        pltpu.make_async_copy(ids4_hbm.at[base], idx4_c[b], isem[b]).wait()
        idxr = idx_c[b]
        pltpu.async_copy(frozen_hbm.at[idxr], frozen_c[b], gsem[b])
        pltpu.async_copy(item_w_hbm.at[idxr], learn_c[b], gsem[b])
        pltpu.async_copy(tags_hbm.at[idx4_c[b]], tags_c[b], gsem[b])

    def waitg(b):
        idxr = idx_c[b]
        pltpu.make_async_copy(frozen_hbm.at[idxr], frozen_c[b], gsem[b]).wait()
        pltpu.make_async_copy(item_w_hbm.at[idxr], learn_c[b], gsem[b]).wait()
        pltpu.make_async_copy(tags_hbm.at[idxr], tags_c[b], gsem[b]).wait()

    def iw(g, b):
        s = base + g
        l = s // _NBT
        bt = s % _NBT
        pltpu.async_copy(slab_c[b].at[:, :, pl.ds(0, _C)],
                         w_hbm.at[l, :, bt], wsem[b])

    def waitw(b):
        pltpu.make_async_copy(slab_c[b].at[:, :, pl.ds(0, _C)],
                              w_hbm.at[0, :, 0], wsem[b]).wait()

    def compute(b):
        fz, ln, tg, W = frozen_c[b], learn_c[b], tags_c[b], slab_c[b]
        ids_v = idx_c[b]

        @plsc.parallel_loop(0, _C, 1, unroll=4)
        def _item(br):
            col = jnp.full((_L,), 0, jnp.int32) + br
            for k in range(8):
                v = fz[br, pl.ds(k * _L, _L)]
                plsc.store_scatter(W, [dtv[k], drv[k], col], v)
            for k in range(4):
                v = ln[br, pl.ds(k * _L, _L)]
                plsc.store_scatter(W, [dtv[8 + k], drv[8 + k], col], v)

        @plsc.parallel_loop(0, _C // _L, 1, unroll=2)
        def _grp(h):
            rows = h * _L + iota
            ids16 = ids_v[pl.ds(h * _L, _L)]
            col0 = lax.shift_left(lax.bitwise_and(ids16, 3), 1)
            w0 = plsc.load_gather(tg, [rows, col0])
            w1 = plsc.load_gather(tg, [rows, col0 + 1])
            mask10 = jnp.full((_L,), 1023, jnp.int32)
            tv = [
                lax.bitwise_and(w0, mask10),
                lax.bitwise_and(lax.shift_right_logical(w0, 10), mask10),
                lax.shift_right_logical(w0, 20),
                lax.bitwise_and(w1, mask10),
                lax.shift_right_logical(w1, 10),
            ]
            for d in range(_TAG_DIM):
                dcol = jnp.full((_L,), d, jnp.int32)
                acc = plsc.load_gather(tag_tab, [tv[0], dcol])
                for t in range(1, _TAGS_PER_ITEM):
                    acc = acc + plsc.load_gather(tag_tab, [tv[t], dcol])
                dd = _FROZEN_DIM + _EMBED_DIM + d
                W[dd >> 3, dd & 7, pl.ds(h * _L, _L)] = acc * inv_tags

    idx_start(0, 0)
    idx_start(1, 1)
    ig(0, 0)

    def step(s, carry):
        g0 = 2 * s
        pl.when(s > 0)(lambda: waitw(1))
        ig(g0 + 1, 1)
        waitg(0)
        compute(0)
        pl.when(s < _NSTEP - 1)(lambda: idx_start(g0 + 2, 0))
        iw(g0, 0)

        def prefetch_even():
            waitw(0)
            ig(g0 + 2, 0)

        pl.when(s < _NSTEP - 1)(prefetch_even)
        waitg(1)
        compute(1)
        pl.when(s < _NSTEP - 1)(lambda: idx_start(g0 + 3, 1))
        iw(g0 + 1, 1)
        return carry

    lax.fori_loop(0, _NSTEP, step, 0)
    waitw(0)
    waitw(1)


_sc_kernel = functools.partial(
    pl.kernel,
    out_type=jax.ShapeDtypeStruct((_H, _NDT, _NBT, 8, _C), jnp.float32),
    mesh=plsc.VectorSubcoreMesh(core_axis_name="c", subcore_axis_name="s",
                                num_cores=_NC, num_subcores=_NS),
    scratch_types=[
        pltpu.VMEM((_C,), jnp.int32),
        pltpu.VMEM((_C,), jnp.int32),
        pltpu.VMEM((_C,), jnp.int32),
        pltpu.VMEM((_C,), jnp.int32),
        pltpu.VMEM((_C, _FROZEN_DIM), jnp.float32),
        pltpu.VMEM((_C, _FROZEN_DIM), jnp.float32),
        pltpu.VMEM((_C, _EMBED_DIM), jnp.float32),
        pltpu.VMEM((_C, _EMBED_DIM), jnp.float32),
        pltpu.VMEM((_C, 8), jnp.int32),
        pltpu.VMEM((_C, 8), jnp.int32),
        pltpu.VMEM((_NDT, 8, _PITCH), jnp.float32),
        pltpu.VMEM((_NDT, 8, _PITCH), jnp.float32),
        pltpu.VMEM((_NUM_TAGS, _TAG_DIM), jnp.float32),
        pltpu.SemaphoreType.DMA,
        pltpu.SemaphoreType.DMA,
        pltpu.SemaphoreType.DMA,
        pltpu.SemaphoreType.DMA,
        pltpu.SemaphoreType.DMA,
        pltpu.SemaphoreType.DMA,
    ],
    compiler_params=pltpu.CompilerParams(use_tc_tiling_on_sc=False,
                                         needs_layout_passes=False),
)(_sc_body)


def kernel(item_ids, frozen_emb, item_tags, item_emb_w, tag_emb_w):
    ids_t = item_ids.astype(jnp.int32).T.reshape(_NSLAB, _C)
    t = item_tags.astype(jnp.int32)
    w0 = t[:, 0] | (t[:, 1] << 10) | (t[:, 2] << 20)
    w1 = t[:, 3] | (t[:, 4] << 10)
    packed = jnp.stack([w0, w1], axis=1).reshape(-1)
    packed = jnp.pad(packed, (0, 192)).reshape((_NUM_ITEMS + 96) // 4, 8)
    ids4 = ids_t >> 2
    w = _sc_kernel(ids_t, ids4, frozen_emb, packed, item_emb_w, tag_emb_w)
    return jnp.transpose(w, (2, 4, 0, 1, 3)).reshape(_B, _H, _OUT_DIM)
